# baseline (speedup 1.0000x reference)
import functools

import jax
import jax.numpy as jnp
from jax import lax
from jax.experimental import pallas as pl
from jax.experimental.pallas import tpu as pltpu
from jax.experimental.pallas import tpu_sc as plsc

VOCAB = 1_000_000
EMBED_DIM = 32
NUM_CLASSES = 16
BATCH = 16384
SEQ = 200

NUM_WORKERS = 32

PROJ_BLOCK = 32768
NPB = pl.cdiv(VOCAB, PROJ_BLOCK)
VPAD = NPB * PROJ_BLOCK

WSLICE = VPAD // NUM_WORKERS
TCH = 496
NCH = WSLICE // TCH
NBUF = 4

ROWS_PER_WORKER = BATCH // NUM_WORKERS
CHUNK_ROWS = 16
IDX_PER_CHUNK = CHUNK_ROWS * SEQ
GATHER_WIDTH = 128
N_GATHERS = IDX_PER_CHUNK // GATHER_WIDTH
CHUNKS = ROWS_PER_WORKER // CHUNK_ROWS

_SC_PARAMS = pltpu.CompilerParams(use_tc_tiling_on_sc=False)


def _proj_body(t_ref, w_ref, *o_refs):
    r_t = lax.dot_general(
        w_ref[...], t_ref[...], (((0,), (0,)), ((), ())),
        preferred_element_type=jnp.float32,
    ) * (1.0 / SEQ)
    for c in range(NUM_CLASSES):
        o_refs[c][...] = r_t[c, :]


def _project(table_t, w):
    return pl.pallas_call(
        _proj_body,
        grid=(NPB,),
        in_specs=[
            pl.BlockSpec((EMBED_DIM, PROJ_BLOCK), lambda i: (0, i)),
            pl.BlockSpec((EMBED_DIM, NUM_CLASSES), lambda i: (0, 0)),
        ],
        out_specs=[
            pl.BlockSpec((PROJ_BLOCK,), lambda i: (i,))
        ] * NUM_CLASSES,
        out_shape=[
            jax.ShapeDtypeStruct((VPAD,), jnp.float32)
        ] * NUM_CLASSES,
    )(table_t, w)


def _tr_body(*refs):
    p_refs = refs[:NUM_CLASSES]
    out_hbm = refs[NUM_CLASSES]
    cls_v = refs[NUM_CLASSES + 1]
    ploc_v = refs[NUM_CLASSES + 2]
    sems = refs[NUM_CLASSES + 3:NUM_CLASSES + 3 + NBUF]
    semst = refs[NUM_CLASSES + 3 + NBUF:]

    wid = lax.axis_index("s") * 2 + lax.axis_index("c")
    base = wid * WSLICE
    base16 = wid * (WSLICE // 16)
    lanes = lax.iota(jnp.int32, 16)

    def fire(k, slot):
        off = base16 + k * (TCH // 16)
        for c in range(NUM_CLASSES):
            pltpu.async_copy(
                p_refs[c].at[pl.ds(off, TCH // 16)], cls_v.at[slot, c],
                sems[slot],
            )

    def drain(k, slot):
        off = base16 + k * (TCH // 16)
        for c in range(NUM_CLASSES):
            pltpu.make_async_copy(
                p_refs[c].at[pl.ds(off, TCH // 16)], cls_v.at[slot, c],
                sems[slot],
            ).wait()

    def store_desc(k, slot):
        return pltpu.make_async_copy(
            ploc_v.at[slot, :, pl.ds(0, NUM_CLASSES)],
            out_hbm.at[pl.ds(base + k * TCH, TCH)],
            semst[slot],
        )

    def work(k, slot):
        @pl.when(k >= NBUF)
        def _():
            store_desc(k - NBUF, slot).wait()

        def group(g, carry):
            rows = lanes + g * 16
            for c in range(NUM_CLASSES):
                v = cls_v[slot, c, g, :]
                plsc.store_scatter(
                    ploc_v.at[slot],
                    [rows, jnp.full((16,), c, jnp.int32)],
                    v,
                )
            return carry

        lax.fori_loop(0, TCH // 16, group, 0)
        store_desc(k, slot).start()

    for s in range(NBUF - 1):
        fire(s, s)

    def outer(kq, carry):
        for slot in range(NBUF):
            k = kq * NBUF + slot

            @pl.when(k < NCH - (NBUF - 1))
            def _():
                fire(k + NBUF - 1, (slot + NBUF - 1) % NBUF)

            drain(k, slot)
            work(k, slot)
        return carry

    lax.fori_loop(0, NCH // NBUF, outer, 0)
    for s in range(NBUF):
        store_desc(NCH - NBUF + s, s).wait()


def _transpose(ps):
    mesh = plsc.VectorSubcoreMesh(core_axis_name="c", subcore_axis_name="s")
    kern = pl.kernel(
        _tr_body,
        out_type=jax.ShapeDtypeStruct((VPAD, NUM_CLASSES), jnp.float32),
        mesh=mesh,
        compiler_params=pltpu.CompilerParams(
            use_tc_tiling_on_sc=False, needs_layout_passes=False
        ),
        scratch_types=(
            [
                pltpu.VMEM((NBUF, NUM_CLASSES, TCH // 16, 16), jnp.float32),
                pltpu.VMEM((NBUF, TCH, NUM_CLASSES + 1), jnp.float32),
            ]
            + [pltpu.SemaphoreType.DMA] * (2 * NBUF)
        ),
    )
    return kern(*[p.reshape(VPAD // 16, 16) for p in ps])


def _sc_body(x_hbm, p_hbm, b_hbm, out_hbm, idx_v, rows_v, out_v, bias_v,
             semr0, semr1, semi0, semi1, semo0, semo1):
    wid = lax.axis_index("s") * 2 + lax.axis_index("c")

    pltpu.sync_copy(b_hbm, bias_v)
    bias = bias_v[...]

    semr = (semr0, semr1)
    semi = (semi0, semi1)
    semo = (semo0, semo1)

    def idx_desc(chunk, slot):
        return pltpu.make_async_copy(
            x_hbm.at[wid * CHUNKS + chunk], idx_v.at[slot], semi[slot]
        )

    def fire_gathers(slot):
        for j in range(N_GATHERS):
            pltpu.async_copy(
                p_hbm.at[idx_v.at[slot, j]],
                rows_v.at[slot, pl.ds(j * GATHER_WIDTH, GATHER_WIDTH)],
                semr[slot],
            )

    def drain_rows(slot):
        pltpu.make_async_copy(
            p_hbm.at[pl.ds(0, IDX_PER_CHUNK)], rows_v.at[slot], semr[slot]
        ).wait()

    def out_desc(chunk, slot):
        row_base = wid * ROWS_PER_WORKER + chunk * CHUNK_ROWS
        return pltpu.make_async_copy(
            out_v.at[slot], out_hbm.at[pl.ds(row_base, CHUNK_ROWS)],
            semo[slot],
        )

    def accumulate_and_store(chunk, slot):
        @pl.when(chunk >= 2)
        def _():
            out_desc(chunk - 2, slot).wait()

        for r in range(CHUNK_ROWS):
            base = r * SEQ

            def body(k, accs):
                a0, a1, a2, a3 = accs
                a0 = a0 + rows_v[slot, base + k, :]
                a1 = a1 + rows_v[slot, base + 50 + k, :]
                a2 = a2 + rows_v[slot, base + 100 + k, :]
                a3 = a3 + rows_v[slot, base + 150 + k, :]
                return a0, a1, a2, a3

            z = jnp.zeros((16,), jnp.float32)
            a0, a1, a2, a3 = lax.fori_loop(0, SEQ // 4, body, (z, z, z, z))
            out_v[slot, r, :] = (a0 + a1) + (a2 + a3) + bias
        row_base = wid * ROWS_PER_WORKER + chunk * CHUNK_ROWS
        pltpu.async_copy(
            out_v.at[slot], out_hbm.at[pl.ds(row_base, CHUNK_ROWS)],
            semo[slot],
        )

    idx_desc(0, 0).start()
    idx_desc(0, 0).wait()
    fire_gathers(0)
    idx_desc(1, 1).start()

    def outer(c2, carry):
        for slot in range(2):
            c = c2 * 2 + slot

            @pl.when(c < CHUNKS - 1)
            def _():
                idx_desc(c + 1, 1 - slot).wait()
                fire_gathers(1 - slot)

            drain_rows(slot)

            @pl.when(c < CHUNKS - 2)
            def _():
                idx_desc(c + 2, slot).start()

            accumulate_and_store(c, slot)
        return carry

    lax.fori_loop(0, CHUNKS // 2, outer, 0)
    out_desc(CHUNKS - 2, 0).wait()
    out_desc(CHUNKS - 1, 1).wait()


def _sc_pool(x_r, p, b):
    mesh = plsc.VectorSubcoreMesh(core_axis_name="c", subcore_axis_name="s")
    kern = pl.kernel(
        _sc_body,
        out_type=jax.ShapeDtypeStruct((BATCH, NUM_CLASSES), jnp.float32),
        mesh=mesh,
        compiler_params=_SC_PARAMS,
        scratch_types=[
            pltpu.VMEM((2, N_GATHERS, GATHER_WIDTH), jnp.int32),
            pltpu.VMEM((2, IDX_PER_CHUNK, NUM_CLASSES), jnp.float32),
            pltpu.VMEM((2, CHUNK_ROWS, NUM_CLASSES), jnp.float32),
            pltpu.VMEM((NUM_CLASSES,), jnp.float32),
            pltpu.SemaphoreType.DMA,
            pltpu.SemaphoreType.DMA,
            pltpu.SemaphoreType.DMA,
            pltpu.SemaphoreType.DMA,
            pltpu.SemaphoreType.DMA,
            pltpu.SemaphoreType.DMA,
        ],
    )
    return kern(x_r, p, b)


def kernel(x, table, W, b):
    ps = _project(table.T, W)
    p = _transpose(ps)
    x_r = x.astype(jnp.int32).reshape(
        NUM_WORKERS * CHUNKS, N_GATHERS, GATHER_WIDTH
    )
    return _sc_pool(x_r, p, b)


# speedup vs baseline: 1.4554x; 1.4554x over previous
import functools

import jax
import jax.numpy as jnp
from jax import lax
from jax.experimental import pallas as pl
from jax.experimental.pallas import tpu as pltpu
from jax.experimental.pallas import tpu_sc as plsc

VOCAB = 1_000_000
EMBED_DIM = 32
NUM_CLASSES = 16
BATCH = 16384
SEQ = 200

NUM_WORKERS = 32

PROJ_BLOCK = 32768
NPB = pl.cdiv(VOCAB, PROJ_BLOCK)
VPAD = NPB * PROJ_BLOCK

WSLICE = VPAD // NUM_WORKERS
TCH = 1984
NCH = WSLICE // TCH
NBUF = 2

ROWS_PER_WORKER = BATCH // NUM_WORKERS
CHUNK_ROWS = 16
IDX_PER_CHUNK = CHUNK_ROWS * SEQ
GATHER_WIDTH = 128
N_GATHERS = IDX_PER_CHUNK // GATHER_WIDTH
CHUNKS = ROWS_PER_WORKER // CHUNK_ROWS

_SC_PARAMS = pltpu.CompilerParams(use_tc_tiling_on_sc=False)


def _proj_body(t_ref, w_ref, *o_refs):
    r_t = lax.dot_general(
        w_ref[...], t_ref[...], (((0,), (0,)), ((), ())),
        preferred_element_type=jnp.float32,
    ) * (1.0 / SEQ)
    for c in range(NUM_CLASSES):
        o_refs[c][...] = r_t[c, :]


def _project(table_t, w):
    return pl.pallas_call(
        _proj_body,
        grid=(NPB,),
        in_specs=[
            pl.BlockSpec((EMBED_DIM, PROJ_BLOCK), lambda i: (0, i)),
            pl.BlockSpec((EMBED_DIM, NUM_CLASSES), lambda i: (0, 0)),
        ],
        out_specs=[
            pl.BlockSpec((PROJ_BLOCK,), lambda i: (i,))
        ] * NUM_CLASSES,
        out_shape=[
            jax.ShapeDtypeStruct((VPAD,), jnp.float32)
        ] * NUM_CLASSES,
    )(table_t, w)


def _tr_body(*refs):
    p_refs = refs[:NUM_CLASSES]
    out_hbm = refs[NUM_CLASSES]
    cls_v = refs[NUM_CLASSES + 1]
    ploc_v = refs[NUM_CLASSES + 2]
    sems = refs[NUM_CLASSES + 3:NUM_CLASSES + 3 + NBUF]
    semst = refs[NUM_CLASSES + 3 + NBUF:]

    wid = lax.axis_index("s") * 2 + lax.axis_index("c")
    base = wid * WSLICE
    base16 = wid * (WSLICE // 16)
    lanes = lax.iota(jnp.int32, 16)

    def fire(k, slot):
        off = base16 + k * (TCH // 16)
        for c in range(NUM_CLASSES):
            pltpu.async_copy(
                p_refs[c].at[pl.ds(off, TCH // 16)], cls_v.at[slot, c],
                sems[slot],
            )

    def drain(k, slot):
        off = base16 + k * (TCH // 16)
        for c in range(NUM_CLASSES):
            pltpu.make_async_copy(
                p_refs[c].at[pl.ds(off, TCH // 16)], cls_v.at[slot, c],
                sems[slot],
            ).wait()

    def store_desc(k, slot):
        return pltpu.make_async_copy(
            ploc_v.at[slot], out_hbm.at[pl.ds(base + k * TCH, TCH)],
            semst[slot],
        )

    def work(k, slot):
        @pl.when(k >= NBUF)
        def _():
            store_desc(k - NBUF, slot).wait()

        def group(g, carry):
            rows = lanes + g * 16
            for c in range(NUM_CLASSES):
                v = cls_v[slot, c, g, :]
                plsc.store_scatter(
                    ploc_v.at[slot],
                    [rows, jnp.full((16,), c, jnp.int32)],
                    v,
                )
            return carry

        lax.fori_loop(0, TCH // 16, group, 0)
        store_desc(k, slot).start()

    for s in range(NBUF - 1):
        fire(s, s)

    def outer(kq, carry):
        for slot in range(NBUF):
            k = kq * NBUF + slot

            @pl.when(k < NCH - (NBUF - 1))
            def _():
                fire(k + NBUF - 1, (slot + NBUF - 1) % NBUF)

            drain(k, slot)
            work(k, slot)
        return carry

    lax.fori_loop(0, NCH // NBUF, outer, 0)
    for s in range(NBUF):
        store_desc(NCH - NBUF + s, s).wait()


def _transpose(ps):
    mesh = plsc.VectorSubcoreMesh(core_axis_name="c", subcore_axis_name="s")
    kern = pl.kernel(
        _tr_body,
        out_type=jax.ShapeDtypeStruct((VPAD, NUM_CLASSES), jnp.float32),
        mesh=mesh,
        compiler_params=pltpu.CompilerParams(
            use_tc_tiling_on_sc=False, needs_layout_passes=False
        ),
        scratch_types=(
            [
                pltpu.VMEM((NBUF, NUM_CLASSES, TCH // 16, 16), jnp.float32),
                pltpu.VMEM((NBUF, TCH, NUM_CLASSES), jnp.float32),
            ]
            + [pltpu.SemaphoreType.DMA] * (2 * NBUF)
        ),
    )
    return kern(*[p.reshape(VPAD // 16, 16) for p in ps])


def _sc_body(x_hbm, p_hbm, b_hbm, out_hbm, idx_v, rows_v, out_v, bias_v,
             semr0, semr1, semi0, semi1, semo0, semo1):
    wid = lax.axis_index("s") * 2 + lax.axis_index("c")

    pltpu.sync_copy(b_hbm, bias_v)
    bias = bias_v[...]

    semr = (semr0, semr1)
    semi = (semi0, semi1)
    semo = (semo0, semo1)

    def idx_desc(chunk, slot):
        return pltpu.make_async_copy(
            x_hbm.at[wid * CHUNKS + chunk], idx_v.at[slot], semi[slot]
        )

    def fire_gathers(slot):
        for j in range(N_GATHERS):
            pltpu.async_copy(
                p_hbm.at[idx_v.at[slot, j]],
                rows_v.at[slot, pl.ds(j * GATHER_WIDTH, GATHER_WIDTH)],
                semr[slot],
            )

    def drain_rows(slot):
        pltpu.make_async_copy(
            p_hbm.at[pl.ds(0, IDX_PER_CHUNK)], rows_v.at[slot], semr[slot]
        ).wait()

    def out_desc(chunk, slot):
        row_base = wid * ROWS_PER_WORKER + chunk * CHUNK_ROWS
        return pltpu.make_async_copy(
            out_v.at[slot], out_hbm.at[pl.ds(row_base, CHUNK_ROWS)],
            semo[slot],
        )

    def accumulate_and_store(chunk, slot):
        @pl.when(chunk >= 2)
        def _():
            out_desc(chunk - 2, slot).wait()

        for r in range(CHUNK_ROWS):
            base = r * SEQ

            def body(k, accs):
                a0, a1, a2, a3 = accs
                a0 = a0 + rows_v[slot, base + k, :]
                a1 = a1 + rows_v[slot, base + 50 + k, :]
                a2 = a2 + rows_v[slot, base + 100 + k, :]
                a3 = a3 + rows_v[slot, base + 150 + k, :]
                return a0, a1, a2, a3

            z = jnp.zeros((16,), jnp.float32)
            a0, a1, a2, a3 = lax.fori_loop(0, SEQ // 4, body, (z, z, z, z))
            out_v[slot, r, :] = (a0 + a1) + (a2 + a3) + bias
        row_base = wid * ROWS_PER_WORKER + chunk * CHUNK_ROWS
        pltpu.async_copy(
            out_v.at[slot], out_hbm.at[pl.ds(row_base, CHUNK_ROWS)],
            semo[slot],
        )

    idx_desc(0, 0).start()
    idx_desc(0, 0).wait()
    fire_gathers(0)
    idx_desc(1, 1).start()

    def outer(c2, carry):
        for slot in range(2):
            c = c2 * 2 + slot

            @pl.when(c < CHUNKS - 1)
            def _():
                idx_desc(c + 1, 1 - slot).wait()
                fire_gathers(1 - slot)

            drain_rows(slot)

            @pl.when(c < CHUNKS - 2)
            def _():
                idx_desc(c + 2, slot).start()

            accumulate_and_store(c, slot)
        return carry

    lax.fori_loop(0, CHUNKS // 2, outer, 0)
    out_desc(CHUNKS - 2, 0).wait()
    out_desc(CHUNKS - 1, 1).wait()


def _sc_pool(x_r, p, b):
    mesh = plsc.VectorSubcoreMesh(core_axis_name="c", subcore_axis_name="s")
    kern = pl.kernel(
        _sc_body,
        out_type=jax.ShapeDtypeStruct((BATCH, NUM_CLASSES), jnp.float32),
        mesh=mesh,
        compiler_params=_SC_PARAMS,
        scratch_types=[
            pltpu.VMEM((2, N_GATHERS, GATHER_WIDTH), jnp.int32),
            pltpu.VMEM((2, IDX_PER_CHUNK, NUM_CLASSES), jnp.float32),
            pltpu.VMEM((2, CHUNK_ROWS, NUM_CLASSES), jnp.float32),
            pltpu.VMEM((NUM_CLASSES,), jnp.float32),
            pltpu.SemaphoreType.DMA,
            pltpu.SemaphoreType.DMA,
            pltpu.SemaphoreType.DMA,
            pltpu.SemaphoreType.DMA,
            pltpu.SemaphoreType.DMA,
            pltpu.SemaphoreType.DMA,
        ],
    )
    return kern(x_r, p, b)


def kernel(x, table, W, b):
    ps = _project(table.T, W)
    p = _transpose(ps)
    x_r = x.astype(jnp.int32).reshape(
        NUM_WORKERS * CHUNKS, N_GATHERS, GATHER_WIDTH
    )
    return _sc_pool(x_r, p, b)
